# Initial kernel scaffold; baseline (speedup 1.0000x reference)
import functools

import jax
import jax.numpy as jnp
from jax import lax
from jax.experimental import pallas as pl
from jax.experimental.pallas import tpu as pltpu
from jax.experimental.pallas import tpu_sc as plsc

N = 10000
E = 320000
D_IN = 128
DIM1 = 128
D_OUT = 64

NC = 2
NS = 16
NW = NC * NS
EPW = E // NW
C = 80
NCHUNK = EPW // C
RPS = N // NS
ZROWS = 125


def _sc_aggregate(h, src3, dst3, w3, D):
    mesh = plsc.VectorSubcoreMesh(core_axis_name="c", subcore_axis_name="s")

    @functools.partial(
        pl.kernel,
        out_type=jax.ShapeDtypeStruct((NC, N, D), jnp.float32),
        mesh=mesh,
        scratch_types=[
            pltpu.VMEM((NCHUNK, C), jnp.int32),
            pltpu.VMEM((NCHUNK, C), jnp.int32),
            pltpu.VMEM((NCHUNK, C), jnp.float32),
            pltpu.VMEM((C, D), jnp.float32),
            pltpu.VMEM((ZROWS, D), jnp.float32),
            pltpu.VMEM_SHARED((N, D), jnp.float32),
            pltpu.SemaphoreType.DMA,
        ],
    )
    def k(h_hbm, src_hbm, dst_hbm, w_hbm, out_hbm,
          src_v, dst_v, w_v, rows, zbuf, acc, sem):
        cid = lax.axis_index("c")
        sid = lax.axis_index("s")
        wid = sid * NC + cid

        pltpu.sync_copy(src_hbm.at[wid], src_v)
        pltpu.sync_copy(dst_hbm.at[wid], dst_v)
        pltpu.sync_copy(w_hbm.at[wid], w_v)

        zv = jnp.zeros((16,), jnp.float32)

        def zrow(i, carry):
            for kk in range(D // 16):
                zbuf[i, pl.ds(kk * 16, 16)] = zv
            return carry

        lax.fori_loop(0, ZROWS, zrow, 0)
        for z in range(RPS // ZROWS):
            pltpu.sync_copy(zbuf, acc.at[pl.ds(sid * RPS + z * ZROWS, ZROWS)])
        plsc.subcore_barrier()

        def chunk(j, carry):
            pltpu.async_copy(h_hbm.at[src_v.at[j]], rows, sem).wait()

            def edge(e, c2):
                wv = jnp.full((16,), w_v[j, e], jnp.float32)
                for kk in range(D // 16):
                    sl = pl.ds(kk * 16, 16)
                    rows[e, sl] = rows[e, sl] * wv
                return c2

            lax.fori_loop(0, C, edge, 0)
            pltpu.sync_copy(rows, acc.at[dst_v.at[j]], add=True)
            return carry

        lax.fori_loop(0, NCHUNK, chunk, 0)
        plsc.subcore_barrier()

        for z in range(RPS // ZROWS):
            sl = pl.ds(sid * RPS + z * ZROWS, ZROWS)
            pltpu.sync_copy(acc.at[sl], out_hbm.at[cid, sl])

    return k(h, src3, dst3, w3)


_MMBLK = 1000


def _mm1(x, W1):
    def body(x_ref, w_ref, o_ref):
        o_ref[...] = jnp.dot(x_ref[...], w_ref[...],
                             preferred_element_type=jnp.float32)

    return pl.pallas_call(
        body,
        grid=(N // _MMBLK,),
        in_specs=[
            pl.BlockSpec((_MMBLK, D_IN), lambda i: (i, 0)),
            pl.BlockSpec((D_IN, DIM1), lambda i: (0, 0)),
        ],
        out_specs=pl.BlockSpec((_MMBLK, DIM1), lambda i: (i, 0)),
        out_shape=jax.ShapeDtypeStruct((N, DIM1), jnp.float32),
    )(x, W1)


def _combine1(p, W2):
    def body(p_ref, w_ref, o_ref):
        s = p_ref[0] + p_ref[1]
        s = jnp.maximum(s, 0.0)
        sq = jnp.sum(s * s, axis=1, keepdims=True)
        s = s * lax.rsqrt(jnp.maximum(sq, 1e-12))
        o_ref[...] = jnp.dot(s, w_ref[...],
                             preferred_element_type=jnp.float32)

    return pl.pallas_call(
        body,
        grid=(N // _MMBLK,),
        in_specs=[
            pl.BlockSpec((NC, _MMBLK, DIM1), lambda i: (0, i, 0)),
            pl.BlockSpec((DIM1, D_OUT), lambda i: (0, 0)),
        ],
        out_specs=pl.BlockSpec((_MMBLK, D_OUT), lambda i: (i, 0)),
        out_shape=jax.ShapeDtypeStruct((N, D_OUT), jnp.float32),
    )(p, W2)


def _combine2(q):
    def body(q_ref, o_ref):
        o_ref[...] = jnp.maximum(q_ref[0] + q_ref[1], 0.0)

    return pl.pallas_call(
        body,
        grid=(N // _MMBLK,),
        in_specs=[pl.BlockSpec((NC, _MMBLK, D_OUT), lambda i: (0, i, 0))],
        out_specs=pl.BlockSpec((_MMBLK, D_OUT), lambda i: (i, 0)),
        out_shape=jax.ShapeDtypeStruct((N, D_OUT), jnp.float32),
    )(q)


def kernel(x, edge_index, edge_weight, W1, W2):
    src3 = edge_index[0].astype(jnp.int32).reshape(NW, NCHUNK, C)
    dst3 = edge_index[1].astype(jnp.int32).reshape(NW, NCHUNK, C)
    w3 = edge_weight.reshape(NW, NCHUNK, C)

    h = _mm1(x, W1)
    p1 = _sc_aggregate(h, src3, dst3, w3, DIM1)
    h2 = _combine1(p1, W2)
    p2 = _sc_aggregate(h2, src3, dst3, w3, D_OUT)
    return _combine2(p2)


# speedup vs baseline: 2.5947x; 2.5947x over previous
import functools

import jax
import jax.numpy as jnp
from jax import lax
from jax.experimental import pallas as pl
from jax.experimental.pallas import tpu as pltpu
from jax.experimental.pallas import tpu_sc as plsc

N = 10000
E = 320000
D_IN = 128
DIM1 = 128
D_OUT = 64

NC = 2
NS = 16
NW = NC * NS
C = 128
NCHUNK = 80
EPW = NCHUNK * C
E_PAD = NW * EPW
G8 = 8
NG = NCHUNK // G8
PAD_N = 10240
RPS = PAD_N // NS
ZROWS = 128


def _sc_aggregate(h, src3, dst3, w3, D):
    mesh = plsc.VectorSubcoreMesh(core_axis_name="c", subcore_axis_name="s")

    @functools.partial(
        pl.kernel,
        out_type=jax.ShapeDtypeStruct((NC, PAD_N, D), jnp.float32),
        mesh=mesh,
        scratch_types=[
            pltpu.VMEM((G8, C), jnp.int32),
            pltpu.VMEM((G8, C), jnp.int32),
            pltpu.VMEM((G8, C), jnp.float32),
            pltpu.VMEM((C, D), jnp.float32),
            pltpu.VMEM((ZROWS, D), jnp.float32),
            pltpu.VMEM_SHARED((PAD_N, D), jnp.float32),
            pltpu.SemaphoreType.DMA,
        ],
    )
    def k(h_hbm, src_hbm, dst_hbm, w_hbm, out_hbm,
          src_v, dst_v, w_v, rows, zbuf, acc, sem):
        cid = lax.axis_index("c")
        sid = lax.axis_index("s")
        wid = sid * NC + cid

        zv = jnp.zeros((16,), jnp.float32)

        def zrow(i, carry):
            for kk in range(D // 16):
                zbuf[i, pl.ds(kk * 16, 16)] = zv
            return carry

        lax.fori_loop(0, ZROWS, zrow, 0)
        for z in range(RPS // ZROWS):
            pltpu.sync_copy(zbuf, acc.at[pl.ds(sid * RPS + z * ZROWS, ZROWS)])
        plsc.subcore_barrier()

        def group(g, carry):
            base = g * G8
            pltpu.sync_copy(src_hbm.at[wid, pl.ds(base, G8)], src_v)
            pltpu.sync_copy(dst_hbm.at[wid, pl.ds(base, G8)], dst_v)
            pltpu.sync_copy(w_hbm.at[wid, pl.ds(base, G8)], w_v)

            def chunk(jj, c1):
                pltpu.async_copy(h_hbm.at[src_v.at[jj]], rows, sem).wait()

                def egroup(q, c2):
                    w16 = w_v[jj, pl.ds(q * 16, 16)]
                    for e16 in range(16):
                        wv = w16.at[jnp.full((16,), e16, jnp.int32)].get(
                            mode="promise_in_bounds")
                        for kk in range(D // 16):
                            sl = pl.ds(kk * 16, 16)
                            rows[q * 16 + e16, sl] = rows[q * 16 + e16, sl] * wv
                    return c2

                lax.fori_loop(0, C // 16, egroup, 0)
                pltpu.sync_copy(rows, acc.at[dst_v.at[jj]], add=True)
                return c1

            lax.fori_loop(0, G8, chunk, 0)
            return carry

        lax.fori_loop(0, NG, group, 0)
        plsc.subcore_barrier()

        for z in range(RPS // ZROWS):
            sl = pl.ds(sid * RPS + z * ZROWS, ZROWS)
            pltpu.sync_copy(acc.at[sl], out_hbm.at[cid, sl])

    return k(h, src3, dst3, w3)


_MMBLK = 1000


def _mm1(x, W1):
    def body(x_ref, w_ref, o_ref):
        o_ref[...] = jnp.dot(x_ref[...], w_ref[...],
                             preferred_element_type=jnp.float32)

    return pl.pallas_call(
        body,
        grid=(N // _MMBLK,),
        in_specs=[
            pl.BlockSpec((_MMBLK, D_IN), lambda i: (i, 0)),
            pl.BlockSpec((D_IN, DIM1), lambda i: (0, 0)),
        ],
        out_specs=pl.BlockSpec((_MMBLK, DIM1), lambda i: (i, 0)),
        out_shape=jax.ShapeDtypeStruct((N, DIM1), jnp.float32),
    )(x, W1)


def _combine1(p, W2):
    def body(p_ref, w_ref, o_ref):
        s = p_ref[0] + p_ref[1]
        s = jnp.maximum(s, 0.0)
        sq = jnp.sum(s * s, axis=1, keepdims=True)
        s = s * lax.rsqrt(jnp.maximum(sq, 1e-12))
        o_ref[...] = jnp.dot(s, w_ref[...],
                             preferred_element_type=jnp.float32)

    w2p = jnp.pad(W2, ((0, 0), (0, DIM1 - D_OUT)))
    return pl.pallas_call(
        body,
        grid=(N // _MMBLK,),
        in_specs=[
            pl.BlockSpec((NC, _MMBLK, DIM1), lambda i: (0, i, 0)),
            pl.BlockSpec((DIM1, DIM1), lambda i: (0, 0)),
        ],
        out_specs=pl.BlockSpec((_MMBLK, DIM1), lambda i: (i, 0)),
        out_shape=jax.ShapeDtypeStruct((N, DIM1), jnp.float32),
    )(p, w2p)


def _combine2(q):
    def body(q_ref, o_ref):
        o_ref[...] = jnp.maximum(q_ref[0, :, :D_OUT] + q_ref[1, :, :D_OUT],
                                 0.0)

    return pl.pallas_call(
        body,
        grid=(N // _MMBLK,),
        in_specs=[pl.BlockSpec((NC, _MMBLK, DIM1), lambda i: (0, i, 0))],
        out_specs=pl.BlockSpec((_MMBLK, D_OUT), lambda i: (i, 0)),
        out_shape=jax.ShapeDtypeStruct((N, D_OUT), jnp.float32),
    )(q)


def kernel(x, edge_index, edge_weight, W1, W2):
    npad = E_PAD - E
    src3 = jnp.concatenate(
        [edge_index[0].astype(jnp.int32), jnp.zeros((npad,), jnp.int32)]
    ).reshape(NW, NCHUNK, C)
    dst3 = jnp.concatenate(
        [edge_index[1].astype(jnp.int32), jnp.zeros((npad,), jnp.int32)]
    ).reshape(NW, NCHUNK, C)
    w3 = jnp.concatenate(
        [edge_weight, jnp.zeros((npad,), jnp.float32)]
    ).reshape(NW, NCHUNK, C)

    h = _mm1(x, W1)
    p1 = _sc_aggregate(h, src3, dst3, w3, DIM1)[:, :N, :]
    h2 = _combine1(p1, W2)
    p2 = _sc_aggregate(h2, src3, dst3, w3, DIM1)[:, :N, :]
    return _combine2(p2)


# speedup vs baseline: 2.9883x; 1.1517x over previous
import functools

import jax
import jax.numpy as jnp
from jax import lax
from jax.experimental import pallas as pl
from jax.experimental.pallas import tpu as pltpu
from jax.experimental.pallas import tpu_sc as plsc

N = 10000
E = 320000
D_IN = 128
DIM1 = 128
D_OUT = 64

NC = 2
NS = 16
NW = NC * NS
C = 128
NCHUNK = 80
EPW = NCHUNK * C
E_PAD = NW * EPW
G8 = 8
NG = NCHUNK // G8
PAD_N = 10240
RPS = PAD_N // NS
ZROWS = 128


def _sc_aggregate(h, src3, dst3, w3, D, d_scale):
    mesh = plsc.VectorSubcoreMesh(core_axis_name="c", subcore_axis_name="s")

    @functools.partial(
        pl.kernel,
        out_type=jax.ShapeDtypeStruct((NC, PAD_N, D), jnp.float32),
        mesh=mesh,
        scratch_types=[
            pltpu.VMEM((G8, C), jnp.int32),
            pltpu.VMEM((G8, C), jnp.int32),
            pltpu.VMEM((G8, C), jnp.float32),
            pltpu.VMEM((C, D), jnp.float32),
            pltpu.VMEM((C, D), jnp.float32),
            pltpu.VMEM_SHARED((PAD_N, D), jnp.float32),
            pltpu.SemaphoreType.DMA,
            pltpu.SemaphoreType.DMA,
        ],
    )
    def k(h_hbm, src_hbm, dst_hbm, w_hbm, out_hbm,
          src_v, dst_v, w_v, rows_a, rows_b, acc, sem_a, sem_b):
        cid = lax.axis_index("c")
        sid = lax.axis_index("s")
        wid = sid * NC + cid

        zv = jnp.zeros((16,), jnp.float32)

        def zrow(i, carry):
            for kk in range(D // 16):
                rows_a[i, pl.ds(kk * 16, 16)] = zv
            return carry

        lax.fori_loop(0, ZROWS, zrow, 0)
        for z in range(RPS // ZROWS):
            pltpu.sync_copy(rows_a,
                            acc.at[pl.ds(sid * RPS + z * ZROWS, ZROWS)])
        plsc.subcore_barrier()

        def scale_and_scatter(jj, rows):
            def egroup(q, c2):
                w16 = w_v[jj, pl.ds(q * 16, 16)]
                for e16 in range(16):
                    wv = w16.at[jnp.full((16,), e16, jnp.int32)].get(
                        mode="promise_in_bounds")
                    for kk in range(d_scale // 16):
                        sl = pl.ds(kk * 16, 16)
                        rows[q * 16 + e16, sl] = rows[q * 16 + e16, sl] * wv
                return c2

            lax.fori_loop(0, C // 16, egroup, 0)
            pltpu.sync_copy(rows, acc.at[dst_v.at[jj]], add=True)

        def group(g, carry):
            base = g * G8
            pltpu.sync_copy(src_hbm.at[wid, pl.ds(base, G8)], src_v)
            pltpu.sync_copy(dst_hbm.at[wid, pl.ds(base, G8)], dst_v)
            pltpu.sync_copy(w_hbm.at[wid, pl.ds(base, G8)], w_v)

            pltpu.async_copy(h_hbm.at[src_v.at[0]], rows_a, sem_a)

            def pair(t, c1):
                ja = 2 * t
                jb = 2 * t + 1
                pltpu.async_copy(h_hbm.at[src_v.at[jb]], rows_b, sem_b)
                pltpu.make_async_copy(h_hbm.at[src_v.at[ja]], rows_a,
                                      sem_a).wait()
                scale_and_scatter(ja, rows_a)

                @pl.when(t < G8 // 2 - 1)
                def _():
                    pltpu.async_copy(h_hbm.at[src_v.at[ja + 2]], rows_a,
                                     sem_a)

                pltpu.make_async_copy(h_hbm.at[src_v.at[jb]], rows_b,
                                      sem_b).wait()
                scale_and_scatter(jb, rows_b)
                return c1

            lax.fori_loop(0, G8 // 2, pair, 0)
            return carry

        lax.fori_loop(0, NG, group, 0)
        plsc.subcore_barrier()

        for z in range(RPS // ZROWS):
            sl = pl.ds(sid * RPS + z * ZROWS, ZROWS)
            pltpu.sync_copy(acc.at[sl], out_hbm.at[cid, sl])

    return k(h, src3, dst3, w3)


_MMBLK = 1000


def _mm1(x, W1):
    def body(x_ref, w_ref, o_ref):
        o_ref[...] = jnp.dot(x_ref[...], w_ref[...],
                             preferred_element_type=jnp.float32)

    return pl.pallas_call(
        body,
        grid=(N // _MMBLK,),
        in_specs=[
            pl.BlockSpec((_MMBLK, D_IN), lambda i: (i, 0)),
            pl.BlockSpec((D_IN, DIM1), lambda i: (0, 0)),
        ],
        out_specs=pl.BlockSpec((_MMBLK, DIM1), lambda i: (i, 0)),
        out_shape=jax.ShapeDtypeStruct((N, DIM1), jnp.float32),
    )(x, W1)


def _combine1(p, W2):
    def body(p_ref, w_ref, o_ref):
        s = p_ref[0] + p_ref[1]
        s = jnp.maximum(s, 0.0)
        sq = jnp.sum(s * s, axis=1, keepdims=True)
        s = s * lax.rsqrt(jnp.maximum(sq, 1e-12))
        o_ref[...] = jnp.dot(s, w_ref[...],
                             preferred_element_type=jnp.float32)

    w2p = jnp.pad(W2, ((0, 0), (0, DIM1 - D_OUT)))
    return pl.pallas_call(
        body,
        grid=(N // _MMBLK,),
        in_specs=[
            pl.BlockSpec((NC, _MMBLK, DIM1), lambda i: (0, i, 0)),
            pl.BlockSpec((DIM1, DIM1), lambda i: (0, 0)),
        ],
        out_specs=pl.BlockSpec((_MMBLK, DIM1), lambda i: (i, 0)),
        out_shape=jax.ShapeDtypeStruct((N, DIM1), jnp.float32),
    )(p, w2p)


def _combine2(q):
    def body(q_ref, o_ref):
        o_ref[...] = jnp.maximum(q_ref[0, :, :D_OUT] + q_ref[1, :, :D_OUT],
                                 0.0)

    return pl.pallas_call(
        body,
        grid=(N // _MMBLK,),
        in_specs=[pl.BlockSpec((NC, _MMBLK, DIM1), lambda i: (0, i, 0))],
        out_specs=pl.BlockSpec((_MMBLK, D_OUT), lambda i: (i, 0)),
        out_shape=jax.ShapeDtypeStruct((N, D_OUT), jnp.float32),
    )(q)


def kernel(x, edge_index, edge_weight, W1, W2):
    npad = E_PAD - E
    src3 = jnp.concatenate(
        [edge_index[0].astype(jnp.int32), jnp.zeros((npad,), jnp.int32)]
    ).reshape(NW, NCHUNK, C)
    dst3 = jnp.concatenate(
        [edge_index[1].astype(jnp.int32), jnp.zeros((npad,), jnp.int32)]
    ).reshape(NW, NCHUNK, C)
    w3 = jnp.concatenate(
        [edge_weight, jnp.zeros((npad,), jnp.float32)]
    ).reshape(NW, NCHUNK, C)

    h = _mm1(x, W1)
    p1 = _sc_aggregate(h, src3, dst3, w3, DIM1, DIM1)[:, :N, :]
    h2 = _combine1(p1, W2)
    p2 = _sc_aggregate(h2, src3, dst3, w3, DIM1, D_OUT)[:, :N, :]
    return _combine2(p2)
